# baseline (speedup 1.0000x reference)
import functools

import jax
import jax.numpy as jnp
from jax import lax
from jax.experimental import pallas as pl
from jax.experimental.pallas import tpu as pltpu
from jax.experimental.pallas import tpu_sc as plsc

B = 16384
BAG = 50
TD = 64
CD = 32
NEMB = 1000000
HALF = NEMB // 2

NC = 2
NS = 16
NW = NC * NS
EPW = B // NW

CHUNK = 8
RPC = CHUNK * BAG
NCHUNK = EPW // CHUNK
CAT_GB = 64


def _make_sc_body(with_cat):
    def _sc_body(*args):
        if with_cat:
            (text_hbm, cat_hbm, ttab_hbm, ctab_hbm, sum_hbm, cemb_hbm,
             idx0, idx1, rows0, rows1, out0, out1, cidx_v, crows_v,
             sem0, sem1, osem) = args
        else:
            (text_hbm, ttab_hbm, sum_hbm,
             idx0, idx1, rows0, rows1, out0, out1,
             sem0, sem1, osem) = args
        c = lax.axis_index("c")
        s = lax.axis_index("s")
        wid = c * NS + s
        base = pl.multiple_of(wid * EPW, EPW)

        if with_cat:
            crow0 = pl.multiple_of(wid * (EPW // CAT_GB), EPW // CAT_GB)
            pltpu.sync_copy(cat_hbm.at[pl.ds(crow0, EPW // CAT_GB)], cidx_v)
            cds = [pltpu.async_copy(ctab_hbm.at[cidx_v.at[j]],
                                    crows_v.at[pl.ds(j * CAT_GB, CAT_GB)],
                                    sem0)
                   for j in range(EPW // CAT_GB)]
            for d in cds:
                d.wait()
            pltpu.sync_copy(crows_v, cemb_hbm.at[pl.ds(base, EPW)])

        def fire(k, ibuf, rbuf, sem):
            ebase = pl.multiple_of(base + k * CHUNK, CHUNK)
            pltpu.sync_copy(text_hbm.at[pl.ds(ebase, CHUNK)], ibuf)
            for e in range(CHUNK):
                pltpu.async_copy(ttab_hbm.at[ibuf.at[e]],
                                 rbuf.at[pl.ds(e * BAG, BAG)], sem)

        def drain(rbuf, sem):
            pltpu.make_async_copy(sum_hbm.at[pl.ds(0, RPC)], rbuf,
                                  sem).wait()

        def reduce_chunk(k, rbuf, obuf):
            def elem_body(e, c2):
                r0 = e * BAG
                for d in range(TD // 16):
                    acc = rbuf[r0, pl.ds(d * 16, 16)]
                    for l in range(1, BAG):
                        acc = acc + rbuf[r0 + l, pl.ds(d * 16, 16)]
                    obuf[e, pl.ds(d * 16, 16)] = acc
                return c2

            lax.fori_loop(0, CHUNK, elem_body, 0)
            ebase = pl.multiple_of(base + k * CHUNK, CHUNK)
            pltpu.async_copy(obuf, sum_hbm.at[pl.ds(ebase, CHUNK)], osem)

        fire(0, idx0, rows0, sem0)

        def pipe_body(m, carry):
            k = pl.multiple_of(m * 2, 2)
            fire(k + 1, idx1, rows1, sem1)
            drain(rows0, sem0)
            reduce_chunk(k, rows0, out0)

            @pl.when(m < NCHUNK // 2 - 1)
            def _():
                fire(k + 2, idx0, rows0, sem0)

            drain(rows1, sem1)
            reduce_chunk(k + 1, rows1, out1)
            pltpu.make_async_copy(out0, sum_hbm.at[pl.ds(0, CHUNK)],
                                  osem).wait()
            pltpu.make_async_copy(out1, sum_hbm.at[pl.ds(0, CHUNK)],
                                  osem).wait()
            return carry

        lax.fori_loop(0, NCHUNK // 2, pipe_body, 0)

    return _sc_body


_COMMON_SCRATCH = [
    pltpu.VMEM((CHUNK, BAG), jnp.int32),
    pltpu.VMEM((CHUNK, BAG), jnp.int32),
    pltpu.VMEM((RPC, TD), jnp.float32),
    pltpu.VMEM((RPC, TD), jnp.float32),
    pltpu.VMEM((CHUNK, TD), jnp.float32),
    pltpu.VMEM((CHUNK, TD), jnp.float32),
]

_MESH = plsc.VectorSubcoreMesh(core_axis_name="c", subcore_axis_name="s")
_PARAMS = pltpu.CompilerParams(use_tc_tiling_on_sc=False)

_sc_gather_cat = functools.partial(
    pl.kernel,
    out_type=[
        jax.ShapeDtypeStruct((B, TD), jnp.float32),
        jax.ShapeDtypeStruct((B, CD), jnp.float32),
    ],
    mesh=_MESH,
    compiler_params=_PARAMS,
    scratch_types=_COMMON_SCRATCH + [
        pltpu.VMEM((EPW // CAT_GB, CAT_GB), jnp.int32),
        pltpu.VMEM((EPW, CD), jnp.float32),
        pltpu.SemaphoreType.DMA,
        pltpu.SemaphoreType.DMA,
        pltpu.SemaphoreType.DMA,
    ],
)(_make_sc_body(True))

_sc_gather_txt = functools.partial(
    pl.kernel,
    out_type=jax.ShapeDtypeStruct((B, TD), jnp.float32),
    mesh=_MESH,
    compiler_params=_PARAMS,
    scratch_types=_COMMON_SCRATCH + [
        pltpu.SemaphoreType.DMA,
        pltpu.SemaphoreType.DMA,
        pltpu.SemaphoreType.DMA,
    ],
)(_make_sc_body(False))


MLP_BLK = 4096


def _mlp_body(s1_ref, s2_ref, tr_ref, c_ref, t_ref, a1_ref, a2_ref, w2_ref,
              w3_ref, b1_ref, b2_ref, b3_ref, o_ref):
    text = t_ref[...]
    cnt = jnp.sum((text != 0).astype(jnp.float32), axis=1, keepdims=True)
    miss2 = jnp.sum((text < HALF).astype(jnp.float32), axis=1, keepdims=True)
    s = s1_ref[...] + s2_ref[...] - miss2 * tr_ref[...]
    t = s / jnp.maximum(cnt, 1.0)
    h = jnp.dot(t, a1_ref[...], preferred_element_type=jnp.float32)
    h = h + jnp.dot(c_ref[...], a2_ref[...],
                    preferred_element_type=jnp.float32)
    h = jnp.maximum(h + b1_ref[...], 0.0)
    h = jnp.maximum(
        jnp.dot(h, w2_ref[...], preferred_element_type=jnp.float32)
        + b2_ref[...], 0.0)
    o_ref[...] = (jnp.dot(h, w3_ref[...], preferred_element_type=jnp.float32)
                  + b3_ref[...])


def _mlp(s1, s2, trow, cemb, text, a1, a2, w2t, w3t, b1, b2, b3):
    grid = B // MLP_BLK
    h1 = b1.shape[-1]
    h2 = b2.shape[-1]
    return pl.pallas_call(
        _mlp_body,
        grid=(grid,),
        in_specs=[
            pl.BlockSpec((MLP_BLK, TD), lambda i: (i, 0)),
            pl.BlockSpec((MLP_BLK, TD), lambda i: (i, 0)),
            pl.BlockSpec((1, TD), lambda i: (0, 0)),
            pl.BlockSpec((MLP_BLK, CD), lambda i: (i, 0)),
            pl.BlockSpec((MLP_BLK, BAG), lambda i: (i, 0)),
            pl.BlockSpec((TD, h1), lambda i: (0, 0)),
            pl.BlockSpec((CD, h1), lambda i: (0, 0)),
            pl.BlockSpec((h1, h2), lambda i: (0, 0)),
            pl.BlockSpec((h2, TD), lambda i: (0, 0)),
            pl.BlockSpec((1, h1), lambda i: (0, 0)),
            pl.BlockSpec((1, h2), lambda i: (0, 0)),
            pl.BlockSpec((1, TD), lambda i: (0, 0)),
        ],
        out_specs=pl.BlockSpec((MLP_BLK, TD), lambda i: (i, 0)),
        out_shape=jax.ShapeDtypeStruct((B, TD), jnp.float32),
    )(s1, s2, trow, cemb, text, a1, a2, w2t, w3t, b1, b2, b3)


def kernel(text, category, text_table, cat_table, W1, b1, W2, b2, W3, b3):
    text = text.astype(jnp.int32)
    category = category.astype(jnp.int32)
    cat2d = category.reshape(B // CAT_GB, CAT_GB)
    t1 = lax.slice(text_table, (0, 0), (HALF, TD))
    t2 = lax.slice(text_table, (HALF, 0), (NEMB, TD))
    trow = lax.slice(text_table, (HALF, 0), (HALF + 1, TD))
    idx1 = jnp.where(text < HALF, text, 0)
    idx2 = jnp.where(text >= HALF, text - HALF, 0)
    s2 = _sc_gather_txt(idx2, t2)
    s1, cemb = _sc_gather_cat(idx1, cat2d, t1, cat_table)
    a1 = W1.T[:TD, :]
    a2 = W1.T[TD:, :]
    return _mlp(s1, s2, trow, cemb, text, a1, a2, W2.T, W3.T,
                b1.reshape(1, -1), b2.reshape(1, -1), b3.reshape(1, -1))


# speedup vs baseline: 19.6914x; 19.6914x over previous
import functools

import jax
import jax.numpy as jnp
from jax import lax
from jax.experimental import pallas as pl
from jax.experimental.pallas import tpu as pltpu
from jax.experimental.pallas import tpu_sc as plsc

B = 16384
BAG = 50
TD = 64
CD = 32

NC = 2
NS = 16
NW = NC * NS
EPW = B // NW

CHUNK = 16
RPC = CHUNK * BAG
GB = 100
NG = RPC // GB
NCHUNK = EPW // CHUNK
CAT_GB = 64
CAT_PASS = 256


def _sc_body(text_hbm, cat_hbm, ttab_hbm, ctab_hbm, sum_hbm, cemb_hbm,
             idx0, idx1, rows0, rows1, out0, out1, cidx_v, crows_v,
             sem0, sem1, osem):
    c = lax.axis_index("c")
    s = lax.axis_index("s")
    wid = c * NS + s
    base = pl.multiple_of(wid * EPW, EPW)

    for h in range(EPW // CAT_PASS):
        hbase = pl.multiple_of(base + h * CAT_PASS, CAT_PASS)
        crow0 = pl.multiple_of(hbase // CAT_GB, CAT_PASS // CAT_GB)
        pltpu.sync_copy(cat_hbm.at[pl.ds(crow0, CAT_PASS // CAT_GB)], cidx_v)
        cds = [pltpu.async_copy(ctab_hbm.at[cidx_v.at[j]],
                                crows_v.at[pl.ds(j * CAT_GB, CAT_GB)], sem0)
               for j in range(CAT_PASS // CAT_GB)]
        for d in cds:
            d.wait()
        pltpu.sync_copy(crows_v, cemb_hbm.at[pl.ds(hbase, CAT_PASS)])

    def fire(k, ibuf, rbuf, sem):
        trow0 = pl.multiple_of((base + k * CHUNK) * BAG // GB, NG)
        pltpu.sync_copy(text_hbm.at[pl.ds(trow0, NG)], ibuf)
        for g in range(NG):
            pltpu.async_copy(ttab_hbm.at[ibuf.at[g]],
                             rbuf.at[pl.ds(g * GB, GB)], sem)

    def drain(rbuf, sem):
        pltpu.make_async_copy(sum_hbm.at[pl.ds(0, RPC)], rbuf, sem).wait()

    def reduce_chunk(k, rbuf, obuf):
        def elem_body(e, c2):
            r0 = e * BAG
            for d in range(TD // 16):
                acc = rbuf[r0, pl.ds(d * 16, 16)]
                for l in range(1, BAG):
                    acc = acc + rbuf[r0 + l, pl.ds(d * 16, 16)]
                obuf[e, pl.ds(d * 16, 16)] = acc
            return c2

        lax.fori_loop(0, CHUNK, elem_body, 0)
        ebase = pl.multiple_of(base + k * CHUNK, CHUNK)
        pltpu.async_copy(obuf, sum_hbm.at[pl.ds(ebase, CHUNK)], osem)

    fire(0, idx0, rows0, sem0)

    def pipe_body(m, carry):
        k = pl.multiple_of(m * 2, 2)
        fire(k + 1, idx1, rows1, sem1)
        drain(rows0, sem0)
        reduce_chunk(k, rows0, out0)

        @pl.when(m < NCHUNK // 2 - 1)
        def _():
            fire(k + 2, idx0, rows0, sem0)

        drain(rows1, sem1)
        reduce_chunk(k + 1, rows1, out1)
        pltpu.make_async_copy(out0, sum_hbm.at[pl.ds(0, CHUNK)], osem).wait()
        pltpu.make_async_copy(out1, sum_hbm.at[pl.ds(0, CHUNK)], osem).wait()
        return carry

    lax.fori_loop(0, NCHUNK // 2, pipe_body, 0)


_sc_gather = functools.partial(
    pl.kernel,
    out_type=[
        jax.ShapeDtypeStruct((B, TD), jnp.float32),
        jax.ShapeDtypeStruct((B, CD), jnp.float32),
    ],
    mesh=plsc.VectorSubcoreMesh(core_axis_name="c", subcore_axis_name="s"),
    compiler_params=pltpu.CompilerParams(use_tc_tiling_on_sc=False),
    scratch_types=[
        pltpu.VMEM((NG, GB), jnp.int32),
        pltpu.VMEM((NG, GB), jnp.int32),
        pltpu.VMEM((RPC, TD), jnp.float32),
        pltpu.VMEM((RPC, TD), jnp.float32),
        pltpu.VMEM((CHUNK, TD), jnp.float32),
        pltpu.VMEM((CHUNK, TD), jnp.float32),
        pltpu.VMEM((CAT_PASS // CAT_GB, CAT_GB), jnp.int32),
        pltpu.VMEM((CAT_PASS, CD), jnp.float32),
        pltpu.SemaphoreType.DMA,
        pltpu.SemaphoreType.DMA,
        pltpu.SemaphoreType.DMA,
    ],
)(_sc_body)


MLP_BLK = 4096


def _mlp_body(s_ref, c_ref, t_ref, a1_ref, a2_ref, w2_ref, w3_ref,
              b1_ref, b2_ref, b3_ref, o_ref):
    cnt = jnp.sum((t_ref[...] != 0).astype(jnp.float32), axis=1,
                  keepdims=True)
    t = s_ref[...] / jnp.maximum(cnt, 1.0)
    h = jnp.dot(t, a1_ref[...], preferred_element_type=jnp.float32)
    h = h + jnp.dot(c_ref[...], a2_ref[...],
                    preferred_element_type=jnp.float32)
    h = jnp.maximum(h + b1_ref[...], 0.0)
    h = jnp.maximum(
        jnp.dot(h, w2_ref[...], preferred_element_type=jnp.float32)
        + b2_ref[...], 0.0)
    o_ref[...] = (jnp.dot(h, w3_ref[...], preferred_element_type=jnp.float32)
                  + b3_ref[...])


def _mlp(sums, cemb, text, a1, a2, w2t, w3t, b1, b2, b3):
    grid = B // MLP_BLK
    h1 = b1.shape[-1]
    h2 = b2.shape[-1]
    return pl.pallas_call(
        _mlp_body,
        grid=(grid,),
        in_specs=[
            pl.BlockSpec((MLP_BLK, TD), lambda i: (i, 0)),
            pl.BlockSpec((MLP_BLK, CD), lambda i: (i, 0)),
            pl.BlockSpec((MLP_BLK, BAG), lambda i: (i, 0)),
            pl.BlockSpec((TD, h1), lambda i: (0, 0)),
            pl.BlockSpec((CD, h1), lambda i: (0, 0)),
            pl.BlockSpec((h1, h2), lambda i: (0, 0)),
            pl.BlockSpec((h2, TD), lambda i: (0, 0)),
            pl.BlockSpec((1, h1), lambda i: (0, 0)),
            pl.BlockSpec((1, h2), lambda i: (0, 0)),
            pl.BlockSpec((1, TD), lambda i: (0, 0)),
        ],
        out_specs=pl.BlockSpec((MLP_BLK, TD), lambda i: (i, 0)),
        out_shape=jax.ShapeDtypeStruct((B, TD), jnp.float32),
    )(sums, cemb, text, a1, a2, w2t, w3t, b1, b2, b3)


def kernel(text, category, text_table, cat_table, W1, b1, W2, b2, W3, b3):
    text = text.astype(jnp.int32)
    category = category.astype(jnp.int32)
    text2d = text.reshape(B * BAG // GB, GB)
    cat2d = category.reshape(B // CAT_GB, CAT_GB)
    sums, cemb = _sc_gather(text2d, cat2d, text_table, cat_table)
    a1 = W1.T[:TD, :]
    a2 = W1.T[TD:, :]
    return _mlp(sums, cemb, text, a1, a2, W2.T, W3.T,
                b1.reshape(1, -1), b2.reshape(1, -1), b3.reshape(1, -1))


# speedup vs baseline: 19.9072x; 1.0110x over previous
import functools

import jax
import jax.numpy as jnp
from jax import lax
from jax.experimental import pallas as pl
from jax.experimental.pallas import tpu as pltpu
from jax.experimental.pallas import tpu_sc as plsc

B = 16384
BAG = 50
TD = 64
CD = 32

NC = 2
NS = 16
NW = NC * NS
EPW = B // NW

CHUNK = 16
RPC = CHUNK * BAG
GB = 100
NG = RPC // GB
NCHUNK = EPW // CHUNK
CAT_GB = 64
CAT_PASS = 256


def _sc_body(text_hbm, cat_hbm, ttab_hbm, ctab_hbm, sum_hbm, cemb_hbm,
             idx0, idx1, rows0, rows1, out0, out1, cidx_v, crows_v,
             sem0, sem1, osem):
    c = lax.axis_index("c")
    s = lax.axis_index("s")
    wid = c * NS + s
    base = pl.multiple_of(wid * EPW, EPW)

    def fire(k, ibuf, rbuf, sem):
        trow0 = pl.multiple_of((base + k * CHUNK) * BAG // GB, NG)
        pltpu.sync_copy(text_hbm.at[pl.ds(trow0, NG)], ibuf)
        for g in range(NG):
            pltpu.async_copy(ttab_hbm.at[ibuf.at[g]],
                             rbuf.at[pl.ds(g * GB, GB)], sem)

    def drain(rbuf, sem):
        pltpu.make_async_copy(sum_hbm.at[pl.ds(0, RPC)], rbuf, sem).wait()

    def reduce_chunk(k, rbuf, obuf):
        def elem_body(e, c2):
            r0 = e * BAG
            for d in range(TD // 16):
                acc = rbuf[r0, pl.ds(d * 16, 16)]
                for l in range(1, BAG):
                    acc = acc + rbuf[r0 + l, pl.ds(d * 16, 16)]
                obuf[e, pl.ds(d * 16, 16)] = acc
            return c2

        lax.fori_loop(0, CHUNK, elem_body, 0)
        ebase = pl.multiple_of(base + k * CHUNK, CHUNK)
        pltpu.async_copy(obuf, sum_hbm.at[pl.ds(ebase, CHUNK)], osem)

    fire(0, idx0, rows0, sem0)

    for h in range(EPW // CAT_PASS):
        hbase = pl.multiple_of(base + h * CAT_PASS, CAT_PASS)
        crow0 = pl.multiple_of(hbase // CAT_GB, CAT_PASS // CAT_GB)
        pltpu.sync_copy(cat_hbm.at[pl.ds(crow0, CAT_PASS // CAT_GB)], cidx_v)
        cds = [pltpu.async_copy(ctab_hbm.at[cidx_v.at[j]],
                                crows_v.at[pl.ds(j * CAT_GB, CAT_GB)], osem)
               for j in range(CAT_PASS // CAT_GB)]
        for d in cds:
            d.wait()
        pltpu.sync_copy(crows_v, cemb_hbm.at[pl.ds(hbase, CAT_PASS)])

    def pipe_body(m, carry):
        k = pl.multiple_of(m * 2, 2)
        fire(k + 1, idx1, rows1, sem1)
        drain(rows0, sem0)
        reduce_chunk(k, rows0, out0)

        @pl.when(m < NCHUNK // 2 - 1)
        def _():
            fire(k + 2, idx0, rows0, sem0)

        drain(rows1, sem1)
        reduce_chunk(k + 1, rows1, out1)
        pltpu.make_async_copy(out0, sum_hbm.at[pl.ds(0, CHUNK)], osem).wait()
        pltpu.make_async_copy(out1, sum_hbm.at[pl.ds(0, CHUNK)], osem).wait()
        return carry

    lax.fori_loop(0, NCHUNK // 2, pipe_body, 0)


_sc_gather = functools.partial(
    pl.kernel,
    out_type=[
        jax.ShapeDtypeStruct((B, TD), jnp.float32),
        jax.ShapeDtypeStruct((B, CD), jnp.float32),
    ],
    mesh=plsc.VectorSubcoreMesh(core_axis_name="c", subcore_axis_name="s"),
    compiler_params=pltpu.CompilerParams(use_tc_tiling_on_sc=False),
    scratch_types=[
        pltpu.VMEM((NG, GB), jnp.int32),
        pltpu.VMEM((NG, GB), jnp.int32),
        pltpu.VMEM((RPC, TD), jnp.float32),
        pltpu.VMEM((RPC, TD), jnp.float32),
        pltpu.VMEM((CHUNK, TD), jnp.float32),
        pltpu.VMEM((CHUNK, TD), jnp.float32),
        pltpu.VMEM((CAT_PASS // CAT_GB, CAT_GB), jnp.int32),
        pltpu.VMEM((CAT_PASS, CD), jnp.float32),
        pltpu.SemaphoreType.DMA,
        pltpu.SemaphoreType.DMA,
        pltpu.SemaphoreType.DMA,
    ],
)(_sc_body)


MLP_BLK = 4096


def _mlp_body(s_ref, c_ref, t_ref, a1_ref, a2_ref, w2_ref, w3_ref,
              b1_ref, b2_ref, b3_ref, o_ref):
    cnt = jnp.sum((t_ref[...] != 0).astype(jnp.float32), axis=1,
                  keepdims=True)
    t = s_ref[...] / jnp.maximum(cnt, 1.0)
    h = jnp.dot(t, a1_ref[...], preferred_element_type=jnp.float32)
    h = h + jnp.dot(c_ref[...], a2_ref[...],
                    preferred_element_type=jnp.float32)
    h = jnp.maximum(h + b1_ref[...], 0.0)
    h = jnp.maximum(
        jnp.dot(h, w2_ref[...], preferred_element_type=jnp.float32)
        + b2_ref[...], 0.0)
    o_ref[...] = (jnp.dot(h, w3_ref[...], preferred_element_type=jnp.float32)
                  + b3_ref[...])


def _mlp(sums, cemb, text, a1, a2, w2t, w3t, b1, b2, b3):
    grid = B // MLP_BLK
    h1 = b1.shape[-1]
    h2 = b2.shape[-1]
    return pl.pallas_call(
        _mlp_body,
        grid=(grid,),
        in_specs=[
            pl.BlockSpec((MLP_BLK, TD), lambda i: (i, 0)),
            pl.BlockSpec((MLP_BLK, CD), lambda i: (i, 0)),
            pl.BlockSpec((MLP_BLK, BAG), lambda i: (i, 0)),
            pl.BlockSpec((TD, h1), lambda i: (0, 0)),
            pl.BlockSpec((CD, h1), lambda i: (0, 0)),
            pl.BlockSpec((h1, h2), lambda i: (0, 0)),
            pl.BlockSpec((h2, TD), lambda i: (0, 0)),
            pl.BlockSpec((1, h1), lambda i: (0, 0)),
            pl.BlockSpec((1, h2), lambda i: (0, 0)),
            pl.BlockSpec((1, TD), lambda i: (0, 0)),
        ],
        out_specs=pl.BlockSpec((MLP_BLK, TD), lambda i: (i, 0)),
        out_shape=jax.ShapeDtypeStruct((B, TD), jnp.float32),
    )(sums, cemb, text, a1, a2, w2t, w3t, b1, b2, b3)


def kernel(text, category, text_table, cat_table, W1, b1, W2, b2, W3, b3):
    text = text.astype(jnp.int32)
    category = category.astype(jnp.int32)
    text2d = text.reshape(B * BAG // GB, GB)
    cat2d = category.reshape(B // CAT_GB, CAT_GB)
    sums, cemb = _sc_gather(text2d, cat2d, text_table, cat_table)
    a1 = W1.T[:TD, :]
    a2 = W1.T[TD:, :]
    return _mlp(sums, cemb, text, a1, a2, W2.T, W3.T,
                b1.reshape(1, -1), b2.reshape(1, -1), b3.reshape(1, -1))
